# baseline (speedup 1.0000x reference)
import functools
import jax
import jax.numpy as jnp
from jax import lax
from jax.experimental import pallas as pl
from jax.experimental.pallas import tpu as pltpu
from jax.experimental.pallas import tpu_sc as plsc

_NC = 2
_NS = 16
_NW = _NC * _NS
_L = 16

_C = 32


def _sc_body(x_hbm, pe_hbm, o_hbm,
             xb0, xb1, xb2, pe0, pe1,
             s_in0, s_in1, s_in2, s_out0, s_out1, s_out2, s_pe0, s_pe1):
    b, s, d = x_hbm.shape
    rows_per_w = s // _NW
    n_chunks = rows_per_w // _C
    wid = lax.axis_index("s") * _NC + lax.axis_index("c")
    s0 = wid * rows_per_w

    xb = [xb0, xb1, xb2]
    pe_bufs = [pe0, pe1]
    s_in = [s_in0, s_in1, s_in2]
    s_out = [s_out0, s_out1, s_out2]
    s_pe = [s_pe0, s_pe1]
    units = [(c, bi) for c in range(n_chunks) for bi in range(b)]
    n_u = len(units)

    pe_h, in_h, out_h = {}, {}, {}
    pe_h[0] = pltpu.async_copy(pe_hbm.at[pl.ds(s0, _C)], pe_bufs[0], s_pe[0])
    if n_chunks > 1:
        pe_h[1] = pltpu.async_copy(
            pe_hbm.at[pl.ds(s0 + _C, _C)], pe_bufs[1], s_pe[1])

    for t in range(n_u + 1):
        if t < n_u:
            c, bi = units[t]
            if t >= 3:
                out_h[t - 3].wait()
            in_h[t] = pltpu.async_copy(
                x_hbm.at[bi, pl.ds(s0 + c * _C, _C)], xb[t % 3], s_in[t % 3])
        if 0 <= t - 1 < n_u:
            c, bi = units[t - 1]
            if bi == 0:
                pe_h[c].wait()
            in_h[t - 1].wait()
            buf, pe_buf = xb[(t - 1) % 3], pe_bufs[c % 2]

            @plsc.parallel_loop(0, _C, 1, unroll=2)
            def add_row(r, buf=buf, pe_buf=pe_buf):
                for j in range(d // _L):
                    sl = pl.ds(j * _L, _L)
                    buf[r, sl] = buf[r, sl] + pe_buf[r, sl]

            out_h[t - 1] = pltpu.async_copy(
                buf, o_hbm.at[bi, pl.ds(s0 + c * _C, _C)], s_out[(t - 1) % 3])
            if bi == b - 1 and c + 2 < n_chunks:
                pe_h[c + 2] = pltpu.async_copy(
                    pe_hbm.at[pl.ds(s0 + (c + 2) * _C, _C)], pe_bufs[c % 2],
                    s_pe[c % 2])

    out_h[n_u - 3].wait()
    out_h[n_u - 2].wait()
    out_h[n_u - 1].wait()


def kernel(x, pos_emb):
    b, s, d = x.shape
    pe = pos_emb[:s]
    mesh = plsc.VectorSubcoreMesh(core_axis_name="c", subcore_axis_name="s")
    k = functools.partial(
        pl.kernel,
        mesh=mesh,
        out_type=jax.ShapeDtypeStruct((b, s, d), x.dtype),
        scratch_types=[
            pltpu.VMEM((_C, d), jnp.float32),
            pltpu.VMEM((_C, d), jnp.float32),
            pltpu.VMEM((_C, d), jnp.float32),
            pltpu.VMEM((_C, d), jnp.float32),
            pltpu.VMEM((_C, d), jnp.float32),
            pltpu.SemaphoreType.DMA,
            pltpu.SemaphoreType.DMA,
            pltpu.SemaphoreType.DMA,
            pltpu.SemaphoreType.DMA,
            pltpu.SemaphoreType.DMA,
            pltpu.SemaphoreType.DMA,
            pltpu.SemaphoreType.DMA,
            pltpu.SemaphoreType.DMA,
        ],
    )(_sc_body)
    return k(x, pe)
